# Initial kernel scaffold; baseline (speedup 1.0000x reference)
import jax
import jax.numpy as jnp
from jax.experimental import pallas as pl


def kernel(x, edge_attr, edge_index, graph_ids, W_pn, b_pn, W_pe1, b_pe1, W_pe2, b_pe2, W_et, b_et, Wih1, bih1, Whh1, bhh1, W_pe_l, b_pe_l, W_pn2, b_pn2, Wih2, bih2, Whh2, bhh2):
    raise NotImplementedError("write your pallas kernel here")


# speedup vs baseline: 5.9599x; 5.9599x over previous
import functools

import jax
import jax.numpy as jnp
from jax import lax
from jax.experimental import pallas as pl
from jax.experimental.pallas import tpu as pltpu
from jax.experimental.pallas import tpu_sc as plsc

N = 10000
E = 320000
G = 256
H = 128
FN = 74
FE = 12

NP = 10240
RW = 80
RST = 25
NG = E // (RW * RST)
NWORK = 32
GPW = NG // NWORK
BE = 2560
NB = E // BE


def _leaky(v):
    return jnp.where(v >= 0, v, 0.01 * v)


def _mesh():
    return plsc.VectorSubcoreMesh(core_axis_name="c", subcore_axis_name="s")


def _node_prep(x, W_pn, b_pn, W_pe1x, w2a):
    def body(x_r, wpn_r, bpn_r, wpx_r, w2a_r, hv_r, xp_r, q_r):
        xv = x_r[...]
        hv = _leaky(lax.dot_general(xv, wpn_r[...], (((1,), (1,)), ((), ())),
                                    preferred_element_type=jnp.float32)
                    + bpn_r[...])
        hv_r[...] = hv
        xp_r[...] = lax.dot_general(xv, wpx_r[...], (((1,), (1,)), ((), ())),
                                    preferred_element_type=jnp.float32)
        q_r[...] = lax.dot_general(hv, w2a_r[...], (((1,), (0,)), ((), ())),
                                   preferred_element_type=jnp.float32)

    return pl.pallas_call(
        body,
        out_shape=(
            jax.ShapeDtypeStruct((N, H), jnp.float32),
            jax.ShapeDtypeStruct((N, H), jnp.float32),
            jax.ShapeDtypeStruct((N, 1), jnp.float32),
        ),
    )(x, W_pn, b_pn.reshape(1, H), W_pe1x, w2a.reshape(H, 1))


def _gather_stage(table, src3d, dst3d, qtab, qtab2=None, bvec=None):
    two = qtab2 is not None
    out_type = [
        jax.ShapeDtypeStruct((E, H), jnp.float32),
        jax.ShapeDtypeStruct((NG, RST, RW), jnp.float32),
    ]
    scratch = [
        pltpu.VMEM((NP,), jnp.float32),
        pltpu.VMEM((RST, RW), jnp.int32),
        pltpu.VMEM((RST, RW), jnp.int32),
        pltpu.VMEM((RW, H), jnp.float32),
        pltpu.VMEM((RST, RW), jnp.float32),
        pltpu.SemaphoreType.DMA,
    ]
    if two:
        out_type.append(jax.ShapeDtypeStruct((NWORK, 1, 16), jnp.float32))
        scratch += [
            pltpu.VMEM((NP,), jnp.float32),
            pltpu.VMEM((16,), jnp.float32),
            pltpu.VMEM((1, 16), jnp.float32),
        ]

    @functools.partial(pl.kernel, mesh=_mesh(), out_type=tuple(out_type),
                       scratch_types=scratch,
                       compiler_params=pltpu.CompilerParams(
                           needs_layout_passes=False))
    def k(*refs):
        if two:
            (tab_h, s_h, d_h, q_h, q2_h, b_h,
             rows_h, sc_h, mx_h,
             qt, sbuf, dbuf, rbuf, obuf, sem, qt2, bbuf, mbuf) = refs
        else:
            (tab_h, s_h, d_h, q_h,
             rows_h, sc_h,
             qt, sbuf, dbuf, rbuf, obuf, sem) = refs
        cid = lax.axis_index("c")
        sid = lax.axis_index("s")
        wid = sid * 2 + cid
        pltpu.sync_copy(q_h, qt)
        if two:
            pltpu.sync_copy(q2_h, qt2)
            pltpu.sync_copy(b_h, bbuf)
            mbuf[0, pl.ds(0, 16)] = jnp.full((16,), -1e30, jnp.float32)

        def stage(t, _):
            gidx = wid * GPW + t
            pltpu.sync_copy(s_h.at[gidx], sbuf)
            pltpu.sync_copy(d_h.at[gidx], dbuf)

            def inner(j, _):
                pltpu.async_copy(tab_h.at[sbuf.at[j]], rbuf, sem).wait()
                pltpu.sync_copy(
                    rbuf, rows_h.at[pl.ds((gidx * RST + j) * RW, RW)])
                for kk in range(RW // 16):
                    iv = dbuf[j, pl.ds(kk * 16, 16)]
                    qv = plsc.load_gather(qt, [iv])
                    if two:
                        iv2 = sbuf[j, pl.ds(kk * 16, 16)]
                        qv2 = plsc.load_gather(qt2, [iv2])
                        lv = qv + qv2 + bbuf[...]
                        lv = jnp.where(lv >= 0, lv, 0.01 * lv)
                        mbuf[0, pl.ds(0, 16)] = jnp.maximum(
                            mbuf[0, pl.ds(0, 16)], lv)
                        obuf[j, pl.ds(kk * 16, 16)] = lv
                    else:
                        obuf[j, pl.ds(kk * 16, 16)] = qv
                return 0

            lax.fori_loop(0, RST, inner, 0)
            pltpu.sync_copy(obuf, sc_h.at[gidx])
            return 0

        lax.fori_loop(0, GPW, stage, 0)
        if two:
            pltpu.sync_copy(mbuf, mx_h.at[wid])

    if two:
        return k(table, src3d, dst3d, qtab, qtab2, bvec)
    return k(table, src3d, dst3d, qtab)


def _edge1(xs, edge_attr, qd, W_pe1e, b_pe1, w2b, b_pe2, W_et, b_et):
    def body(xs_r, ea_r, qd_r, wpe_r, bp1_r, w2b_r, bp2_r, wet_r, bet_r,
             et_r, lg_r, bm_r):
        ep = lax.dot_general(ea_r[...], wpe_r[...], (((1,), (1,)), ((), ())),
                             preferred_element_type=jnp.float32)
        he1 = _leaky(xs_r[...] + ep + bp1_r[...])
        r = lax.dot_general(he1, w2b_r[...], (((1,), (0,)), ((), ())),
                            preferred_element_type=jnp.float32)
        lg = _leaky(qd_r[...] + r + bp2_r[...])
        lg_r[...] = lg
        et_r[...] = lax.dot_general(he1, wet_r[...], (((1,), (1,)), ((), ())),
                                    preferred_element_type=jnp.float32) + bet_r[...]
        bm_r[...] = jnp.full((1, 1, 128), jnp.max(lg), jnp.float32)

    full = lambda shape: pl.BlockSpec(shape, lambda i: (0, 0))
    return pl.pallas_call(
        body,
        grid=(NB,),
        in_specs=[
            pl.BlockSpec((BE, H), lambda i: (i, 0)),
            pl.BlockSpec((BE, FE), lambda i: (i, 0)),
            pl.BlockSpec((BE, 1), lambda i: (i, 0)),
            full((H, FE)), full((1, H)), full((H, 1)), full((1, 1)),
            full((H, H)), full((1, H)),
        ],
        out_specs=[
            pl.BlockSpec((BE, H), lambda i: (i, 0)),
            pl.BlockSpec((BE, 1), lambda i: (i, 0)),
            pl.BlockSpec((1, 1, 128), lambda i: (i, 0, 0)),
        ],
        out_shape=(
            jax.ShapeDtypeStruct((E, H), jnp.float32),
            jax.ShapeDtypeStruct((E, 1), jnp.float32),
            jax.ShapeDtypeStruct((NB, 1, 128), jnp.float32),
        ),
    )(xs, edge_attr, qd, W_pe1e, b_pe1.reshape(1, H), w2b.reshape(H, 1),
      b_pe2.reshape(1, 1), W_et, b_et.reshape(1, H))


def _edge_softmax_sc(l3d, dst3d, gvec, z1):
    GPS = NG // 16

    @functools.partial(
        pl.kernel, mesh=_mesh(),
        compiler_params=pltpu.CompilerParams(needs_layout_passes=False),
        out_type=jax.ShapeDtypeStruct((NG, RST, RW), jnp.float32),
        scratch_types=[
            pltpu.VMEM((16,), jnp.float32),
            pltpu.VMEM((RST, RW), jnp.float32),
            pltpu.VMEM((RST, RW), jnp.int32),
            pltpu.VMEM((RW,), jnp.float32),
            pltpu.VMEM((NP,), jnp.float32),
            pltpu.VMEM((RST, RW), jnp.float32),
            pltpu.VMEM_SHARED((NP,), jnp.float32),
        ])
    def k(l_h, d_h, g_h, z_h, a_h, gbuf, lbuf, dbuf, ebuf, stab, abuf, s_sh):
        cid = lax.axis_index("c")
        sid = lax.axis_index("s")
        pltpu.sync_copy(g_h, gbuf)
        pltpu.sync_copy(z_h, s_sh.at[pl.ds(sid * (NP // 16), NP // 16)])
        plsc.subcore_barrier()
        g = gbuf[...]

        def stage1(t, _):
            gidx = sid * GPS + t
            pltpu.sync_copy(l_h.at[gidx], lbuf)
            pltpu.sync_copy(d_h.at[gidx], dbuf)

            def inner(j, _):
                for kk in range(RW // 16):
                    lv = lbuf[j, pl.ds(kk * 16, 16)]
                    ebuf[pl.ds(kk * 16, 16)] = jnp.exp(lv - g)
                pltpu.sync_copy(ebuf, s_sh.at[dbuf.at[j]], add=True)
                return 0

            lax.fori_loop(0, RST, inner, 0)
            return 0

        lax.fori_loop(0, GPS, stage1, 0)
        plsc.subcore_barrier()
        pltpu.sync_copy(s_sh, stab)
        wid = sid * 2 + cid

        def stage2(t, _):
            gidx = wid * GPW + t
            pltpu.sync_copy(l_h.at[gidx], lbuf)
            pltpu.sync_copy(d_h.at[gidx], dbuf)

            def inner(j, _):
                for kk in range(RW // 16):
                    lv = lbuf[j, pl.ds(kk * 16, 16)]
                    iv = dbuf[j, pl.ds(kk * 16, 16)]
                    sv = plsc.load_gather(stab, [iv])
                    abuf[j, pl.ds(kk * 16, 16)] = jnp.exp(lv - g) / sv
                return 0

            lax.fori_loop(0, RST, inner, 0)
            pltpu.sync_copy(abuf, a_h.at[gidx])
            return 0

        lax.fori_loop(0, GPW, stage2, 0)

    return k(l3d, dst3d, gvec, z1)


def _scale(a, rows):
    def body(a_r, r_r, m_r):
        m_r[...] = a_r[...] * r_r[...]

    return pl.pallas_call(
        body,
        grid=(NB,),
        in_specs=[pl.BlockSpec((BE, 1), lambda i: (i, 0)),
                  pl.BlockSpec((BE, H), lambda i: (i, 0))],
        out_specs=pl.BlockSpec((BE, H), lambda i: (i, 0)),
        out_shape=jax.ShapeDtypeStruct((E, H), jnp.float32),
    )(a, rows)


def _scatter_stage(m, dst3d, zrows):
    @functools.partial(
        pl.kernel, mesh=_mesh(),
        compiler_params=pltpu.CompilerParams(needs_layout_passes=False),
        out_type=jax.ShapeDtypeStruct((2, NP, H), jnp.float32),
        scratch_types=[
            pltpu.VMEM((RW, H), jnp.float32),
            pltpu.VMEM((RST, RW), jnp.int32),
            pltpu.VMEM_SHARED((NP, H), jnp.float32),
        ])
    def k(m_h, d_h, z_h, cp_h, mbuf, dbuf, csh):
        cid = lax.axis_index("c")
        sid = lax.axis_index("s")
        wid = sid * 2 + cid
        nrow = NP // 16
        pltpu.sync_copy(z_h, csh.at[pl.ds(sid * nrow, nrow)])
        plsc.subcore_barrier()

        def stage(t, _):
            gidx = wid * GPW + t
            pltpu.sync_copy(d_h.at[gidx], dbuf)

            def inner(j, _):
                pltpu.sync_copy(
                    m_h.at[pl.ds((gidx * RST + j) * RW, RW)], mbuf)
                pltpu.sync_copy(mbuf, csh.at[dbuf.at[j]], add=True)
                return 0

            lax.fori_loop(0, RST, inner, 0)
            return 0

        lax.fori_loop(0, GPW, stage, 0)
        plsc.subcore_barrier()
        pltpu.sync_copy(csh.at[pl.ds(sid * nrow, nrow)],
                        cp_h.at[cid, pl.ds(sid * nrow, nrow)])

    return k(m, dst3d, zrows)


def _gru_block(xg, h, Wih, bih, Whh, bhh):
    gi = lax.dot_general(xg, Wih, (((1,), (1,)), ((), ())),
                         preferred_element_type=jnp.float32) + bih
    gh = lax.dot_general(h, Whh, (((1,), (1,)), ((), ())),
                         preferred_element_type=jnp.float32) + bhh
    i_r, i_z, i_n = gi[:, :H], gi[:, H:2 * H], gi[:, 2 * H:]
    h_r, h_z, h_n = gh[:, :H], gh[:, H:2 * H], gh[:, 2 * H:]
    r = jax.nn.sigmoid(i_r + h_r)
    z = jax.nn.sigmoid(i_z + h_z)
    n = jnp.tanh(i_n + r * h_n)
    return (1.0 - z) * n + z * h


def _elu(v):
    return jnp.where(v > 0, v, jnp.exp(v) - 1.0)


def _gru1(cp, hv, Wih1, bih1, Whh1, bhh1, W_pn2, b_pn2, wla, wlb):
    NBN = 5
    BN = N // NBN

    def body(cp_r, hv_r, wih_r, bih_r, whh_r, bhh_r, wpn_r, bpn_r,
             wla_r, wlb_r, h1_r, hp_r, qa_r, qb_r):
        c = cp_r[0] + cp_r[1]
        h1 = jax.nn.relu(_gru_block(_elu(c), hv_r[...], wih_r[...], bih_r[...],
                                    whh_r[...], bhh_r[...]))
        h1_r[...] = h1
        hp_r[...] = lax.dot_general(h1, wpn_r[...], (((1,), (1,)), ((), ())),
                                    preferred_element_type=jnp.float32) + bpn_r[...]
        qa_r[...] = lax.dot_general(h1, wla_r[...], (((1,), (0,)), ((), ())),
                                    preferred_element_type=jnp.float32)
        qb_r[...] = lax.dot_general(h1, wlb_r[...], (((1,), (0,)), ((), ())),
                                    preferred_element_type=jnp.float32)

    full2 = lambda shape: pl.BlockSpec(shape, lambda i: (0, 0))
    return pl.pallas_call(
        body,
        grid=(NBN,),
        in_specs=[
            pl.BlockSpec((2, BN, H), lambda i: (0, i, 0)),
            pl.BlockSpec((BN, H), lambda i: (i, 0)),
            full2((3 * H, H)), full2((1, 3 * H)),
            full2((3 * H, H)), full2((1, 3 * H)),
            full2((H, H)), full2((1, H)),
            full2((H, 1)), full2((H, 1)),
        ],
        out_specs=[
            pl.BlockSpec((BN, H), lambda i: (i, 0)),
            pl.BlockSpec((BN, H), lambda i: (i, 0)),
            pl.BlockSpec((BN, 1), lambda i: (i, 0)),
            pl.BlockSpec((BN, 1), lambda i: (i, 0)),
        ],
        out_shape=(
            jax.ShapeDtypeStruct((N, H), jnp.float32),
            jax.ShapeDtypeStruct((N, H), jnp.float32),
            jax.ShapeDtypeStruct((N, 1), jnp.float32),
            jax.ShapeDtypeStruct((N, 1), jnp.float32),
        ),
    )(cp, hv, Wih1, bih1.reshape(1, 3 * H), Whh1, bhh1.reshape(1, 3 * H),
      W_pn2, b_pn2.reshape(1, H), wla.reshape(H, 1), wlb.reshape(H, 1))


def _gru2_pool(c2p, h1, Wih2, bih2, Whh2, bhh2, gids):
    NBN = 10
    BN = N // NBN

    def body(cp_r, h1_r, wih_r, bih_r, whh_r, bhh_r, gid_r, out_r, cnt_r):
        i = pl.program_id(0)

        @pl.when(i == 0)
        def _():
            out_r[...] = jnp.zeros_like(out_r)
            cnt_r[...] = jnp.zeros_like(cnt_r)

        c = cp_r[0] + cp_r[1]
        h2 = jax.nn.relu(_gru_block(_elu(c), h1_r[...], wih_r[...], bih_r[...],
                                    whh_r[...], bhh_r[...]))
        onehot = (gid_r[...] == lax.broadcasted_iota(jnp.int32, (BN, G), 1)
                  ).astype(jnp.float32)
        out_r[...] += lax.dot_general(onehot, h2, (((0,), (0,)), ((), ())),
                                      preferred_element_type=jnp.float32)
        cnt_r[...] += lax.dot_general(
            onehot, jnp.ones((BN, 1), jnp.float32),
            (((0,), (0,)), ((), ())), preferred_element_type=jnp.float32)

        @pl.when(i == NBN - 1)
        def _():
            out_r[...] = out_r[...] / jnp.maximum(cnt_r[...], 1.0)

    full2 = lambda shape: pl.BlockSpec(shape, lambda i: (0, 0))
    return pl.pallas_call(
        body,
        grid=(NBN,),
        in_specs=[
            pl.BlockSpec((2, BN, H), lambda i: (0, i, 0)),
            pl.BlockSpec((BN, H), lambda i: (i, 0)),
            full2((3 * H, H)), full2((1, 3 * H)),
            full2((3 * H, H)), full2((1, 3 * H)),
            pl.BlockSpec((BN, 1), lambda i: (i, 0)),
        ],
        out_specs=pl.BlockSpec((G, H), lambda i: (0, 0)),
        out_shape=jax.ShapeDtypeStruct((G, H), jnp.float32),
        scratch_shapes=[pltpu.VMEM((G, 1), jnp.float32)],
    )(c2p, h1, Wih2, bih2.reshape(1, 3 * H), Whh2, bhh2.reshape(1, 3 * H),
      gids.reshape(N, 1))


def kernel(x, edge_attr, edge_index, graph_ids,
           W_pn, b_pn, W_pe1, b_pe1, W_pe2, b_pe2, W_et, b_et,
           Wih1, bih1, Whh1, bhh1,
           W_pe_l, b_pe_l, W_pn2, b_pn2, Wih2, bih2, Whh2, bhh2):
    src3d = edge_index[0].reshape(NG, RST, RW)
    dst3d = edge_index[1].reshape(NG, RST, RW)

    W_pe1x = W_pe1[:, :FN]
    W_pe1e = W_pe1[:, FN:]
    w2a = W_pe2[0, :H]
    w2b = W_pe2[0, H:]
    wla = W_pe_l[0, :H]
    wlb = W_pe_l[0, H:]

    z1 = jnp.zeros((NP // 16,), jnp.float32)
    zrows = jnp.zeros((NP // 16, H), jnp.float32)

    def padN(v):
        return jnp.concatenate([v.reshape(N), jnp.zeros((NP - N,), jnp.float32)])

    hv, xp, q = _node_prep(x, W_pn, b_pn, W_pe1x, w2a)
    xs, qd3d = _gather_stage(xp, src3d, dst3d, padN(q))
    et1, logits, bmax = _edge1(xs, edge_attr, qd3d.reshape(E, 1),
                               W_pe1e, b_pe1, w2b, b_pe2, W_et, b_et)
    g1 = jnp.full((16,), jnp.max(bmax), jnp.float32)
    a1 = _edge_softmax_sc(logits.reshape(NG, RST, RW), dst3d, g1, z1)
    m1 = _scale(a1.reshape(E, 1), et1)
    cp = _scatter_stage(m1, dst3d, zrows)

    h1, hp, qa, qb = _gru1(cp, hv, Wih1, bih1, Whh1, bhh1,
                           W_pn2, b_pn2, wla, wlb)
    bl = jnp.full((16,), b_pe_l[0], jnp.float32)
    xs2, l2, wmax = _gather_stage(hp, src3d, dst3d, padN(qa), padN(qb), bl)
    g2 = jnp.full((16,), jnp.max(wmax), jnp.float32)
    a2 = _edge_softmax_sc(l2, dst3d, g2, z1)
    m2 = _scale(a2.reshape(E, 1), xs2)
    c2p = _scatter_stage(m2, dst3d, zrows)

    return _gru2_pool(c2p, h1, Wih2, bih2, Whh2, bhh2, graph_ids)


# speedup vs baseline: 9.5451x; 1.6015x over previous
import functools

import jax
import jax.numpy as jnp
from jax import lax
from jax.experimental import pallas as pl
from jax.experimental.pallas import tpu as pltpu
from jax.experimental.pallas import tpu_sc as plsc

N = 10000
E = 320000
G = 256
H = 128
FN = 74
FE = 12

NP = 10240
RW = 80
RST = 25
NG = E // (RW * RST)
NWORK = 32
GPW = NG // NWORK
BE = 2560
NB = E // BE

_SC_PARAMS = pltpu.CompilerParams(needs_layout_passes=False)


def _leaky(v):
    return jnp.where(v >= 0, v, 0.01 * v)


def _mesh():
    return plsc.VectorSubcoreMesh(core_axis_name="c", subcore_axis_name="s")


def _node_prep(x, W_pn, b_pn, W_pe1x, w2a):
    def body(x_r, wpn_r, bpn_r, wpx_r, w2a_r, hv_r, xp_r, q_r):
        xv = x_r[...]
        hv = _leaky(lax.dot_general(xv, wpn_r[...], (((1,), (1,)), ((), ())),
                                    preferred_element_type=jnp.float32)
                    + bpn_r[...])
        hv_r[...] = hv
        xp_r[...] = lax.dot_general(xv, wpx_r[...], (((1,), (1,)), ((), ())),
                                    preferred_element_type=jnp.float32)
        q_r[...] = lax.dot_general(hv, w2a_r[...], (((1,), (0,)), ((), ())),
                                   preferred_element_type=jnp.float32)

    return pl.pallas_call(
        body,
        out_shape=(
            jax.ShapeDtypeStruct((N, H), jnp.float32),
            jax.ShapeDtypeStruct((N, H), jnp.float32),
            jax.ShapeDtypeStruct((N, 1), jnp.float32),
        ),
    )(x, W_pn, b_pn.reshape(1, H), W_pe1x, w2a.reshape(H, 1))


def _gather1(table, src3d, dst3d, qtab):
    @functools.partial(
        pl.kernel, mesh=_mesh(), compiler_params=_SC_PARAMS,
        out_type=(
            jax.ShapeDtypeStruct((E, H), jnp.float32),
            jax.ShapeDtypeStruct((NG, RST, RW), jnp.float32),
        ),
        scratch_types=[
            pltpu.VMEM((NP,), jnp.float32),
            pltpu.VMEM((RST, RW), jnp.int32),
            pltpu.VMEM((RST, RW), jnp.int32),
            pltpu.VMEM((RW, H), jnp.float32),
            pltpu.VMEM((RST, RW), jnp.float32),
            pltpu.SemaphoreType.DMA,
        ])
    def k(tab_h, s_h, d_h, q_h, rows_h, sc_h, qt, sbuf, dbuf, rbuf, obuf, sem):
        cid = lax.axis_index("c")
        sid = lax.axis_index("s")
        wid = sid * 2 + cid
        pltpu.sync_copy(q_h, qt)

        def stage(t, _):
            gidx = wid * GPW + t
            pltpu.sync_copy(s_h.at[gidx], sbuf)
            pltpu.sync_copy(d_h.at[gidx], dbuf)

            def inner(j, _):
                pltpu.async_copy(tab_h.at[sbuf.at[j]], rbuf, sem).wait()
                pltpu.sync_copy(
                    rbuf, rows_h.at[pl.ds((gidx * RST + j) * RW, RW)])
                for kk in range(RW // 16):
                    iv = dbuf[j, pl.ds(kk * 16, 16)]
                    obuf[j, pl.ds(kk * 16, 16)] = plsc.load_gather(qt, [iv])
                return 0

            lax.fori_loop(0, RST, inner, 0)
            pltpu.sync_copy(obuf, sc_h.at[gidx])
            return 0

        lax.fori_loop(0, GPW, stage, 0)

    return k(table, src3d, dst3d, qtab)


def _edge1(xs, edge_attr, qd, W_pe1e, b_pe1, w2b, b_pe2):
    def body(xs_r, ea_r, qd_r, wpe_r, bp1_r, w2b_r, bp2_r, he_r, el_r):
        ep = lax.dot_general(ea_r[...], wpe_r[...], (((1,), (1,)), ((), ())),
                             preferred_element_type=jnp.float32)
        he1 = _leaky(xs_r[...] + ep + bp1_r[...])
        r = lax.dot_general(he1, w2b_r[...], (((1,), (0,)), ((), ())),
                            preferred_element_type=jnp.float32)
        e = jnp.exp(_leaky(qd_r[...] + r + bp2_r[...]))
        el_r[...] = e
        he_r[...] = he1 * e

    full = lambda shape: pl.BlockSpec(shape, lambda i: (0, 0))
    return pl.pallas_call(
        body,
        grid=(NB,),
        in_specs=[
            pl.BlockSpec((BE, H), lambda i: (i, 0)),
            pl.BlockSpec((BE, FE), lambda i: (i, 0)),
            pl.BlockSpec((BE, 1), lambda i: (i, 0)),
            full((H, FE)), full((1, H)), full((H, 1)), full((1, 1)),
        ],
        out_specs=[
            pl.BlockSpec((BE, H), lambda i: (i, 0)),
            pl.BlockSpec((BE, 1), lambda i: (i, 0)),
        ],
        out_shape=(
            jax.ShapeDtypeStruct((E, H), jnp.float32),
            jax.ShapeDtypeStruct((E, 1), jnp.float32),
        ),
    )(xs, edge_attr, qd, W_pe1e, b_pe1.reshape(1, H), w2b.reshape(H, 1),
      b_pe2.reshape(1, 1))


def _scatter_l1(rows, el3d, dst3d, zrows, z1):
    @functools.partial(
        pl.kernel, mesh=_mesh(), compiler_params=_SC_PARAMS,
        out_type=(
            jax.ShapeDtypeStruct((2, NP, H), jnp.float32),
            jax.ShapeDtypeStruct((2, NP), jnp.float32),
        ),
        scratch_types=[
            pltpu.VMEM((RW, H), jnp.float32),
            pltpu.VMEM((RST, RW), jnp.int32),
            pltpu.VMEM((RST, RW), jnp.float32),
            pltpu.VMEM_SHARED((NP, H), jnp.float32),
            pltpu.VMEM_SHARED((NP,), jnp.float32),
        ])
    def k(m_h, e_h, d_h, zr_h, z1_h, cp_h, sp_h, mbuf, dbuf, ebuf, csh, ssh):
        cid = lax.axis_index("c")
        sid = lax.axis_index("s")
        wid = sid * 2 + cid
        nrow = NP // 16
        pltpu.sync_copy(zr_h, csh.at[pl.ds(sid * nrow, nrow)])
        pltpu.sync_copy(z1_h, ssh.at[pl.ds(sid * nrow, nrow)])
        plsc.subcore_barrier()

        def stage(t, _):
            gidx = wid * GPW + t
            pltpu.sync_copy(d_h.at[gidx], dbuf)
            pltpu.sync_copy(e_h.at[gidx], ebuf)

            def inner(j, _):
                pltpu.sync_copy(
                    m_h.at[pl.ds((gidx * RST + j) * RW, RW)], mbuf)
                pltpu.sync_copy(mbuf, csh.at[dbuf.at[j]], add=True)
                pltpu.sync_copy(ebuf.at[j], ssh.at[dbuf.at[j]], add=True)
                return 0

            lax.fori_loop(0, RST, inner, 0)
            return 0

        lax.fori_loop(0, GPW, stage, 0)
        plsc.subcore_barrier()
        pltpu.sync_copy(csh.at[pl.ds(sid * nrow, nrow)],
                        cp_h.at[cid, pl.ds(sid * nrow, nrow)])
        pltpu.sync_copy(ssh.at[pl.ds(sid * nrow, nrow)],
                        sp_h.at[cid, pl.ds(sid * nrow, nrow)])

    return k(rows, el3d, dst3d, zrows, z1)


def _gru_block(xg, h, Wih, bih, Whh, bhh):
    gi = lax.dot_general(xg, Wih, (((1,), (1,)), ((), ())),
                         preferred_element_type=jnp.float32) + bih
    gh = lax.dot_general(h, Whh, (((1,), (1,)), ((), ())),
                         preferred_element_type=jnp.float32) + bhh
    i_r, i_z, i_n = gi[:, :H], gi[:, H:2 * H], gi[:, 2 * H:]
    h_r, h_z, h_n = gh[:, :H], gh[:, H:2 * H], gh[:, 2 * H:]
    r = jax.nn.sigmoid(i_r + h_r)
    z = jax.nn.sigmoid(i_z + h_z)
    n = jnp.tanh(i_n + r * h_n)
    return (1.0 - z) * n + z * h


def _elu(v):
    return jnp.where(v > 0, v, jnp.exp(v) - 1.0)


def _ctx(wp_r, sp_r, wmat_r, bias_r):
    s = sp_r[0] + sp_r[1]
    inv = jnp.where(s > 0, 1.0 / jnp.maximum(s, 1e-30), 0.0)
    msk = (s > 0).astype(jnp.float32)
    w = (wp_r[0] + wp_r[1]) * inv
    return lax.dot_general(w, wmat_r, (((1,), (1,)), ((), ())),
                           preferred_element_type=jnp.float32) + msk * bias_r


def _gru1(wp, sp, hv, W_et, b_et, Wih1, bih1, Whh1, bhh1, wla, wlb):
    NBN = 5
    BN = N // NBN

    def body(wp_r, sp_r, hv_r, wet_r, bet_r, wih_r, bih_r, whh_r, bhh_r,
             wla_r, wlb_r, h1_r, qa_r, qb_r):
        c = _ctx(wp_r[...], sp_r[...], wet_r[...], bet_r[...])
        h1 = jax.nn.relu(_gru_block(_elu(c), hv_r[...], wih_r[...], bih_r[...],
                                    whh_r[...], bhh_r[...]))
        h1_r[...] = h1
        qa_r[...] = lax.dot_general(h1, wla_r[...], (((1,), (0,)), ((), ())),
                                    preferred_element_type=jnp.float32)
        qb_r[...] = lax.dot_general(h1, wlb_r[...], (((1,), (0,)), ((), ())),
                                    preferred_element_type=jnp.float32)

    full2 = lambda shape: pl.BlockSpec(shape, lambda i: (0, 0))
    return pl.pallas_call(
        body,
        grid=(NBN,),
        in_specs=[
            pl.BlockSpec((2, BN, H), lambda i: (0, i, 0)),
            pl.BlockSpec((2, BN, 1), lambda i: (0, i, 0)),
            pl.BlockSpec((BN, H), lambda i: (i, 0)),
            full2((H, H)), full2((1, H)),
            full2((3 * H, H)), full2((1, 3 * H)),
            full2((3 * H, H)), full2((1, 3 * H)),
            full2((H, 1)), full2((H, 1)),
        ],
        out_specs=[
            pl.BlockSpec((BN, H), lambda i: (i, 0)),
            pl.BlockSpec((BN, 1), lambda i: (i, 0)),
            pl.BlockSpec((BN, 1), lambda i: (i, 0)),
        ],
        out_shape=(
            jax.ShapeDtypeStruct((N, H), jnp.float32),
            jax.ShapeDtypeStruct((N, 1), jnp.float32),
            jax.ShapeDtypeStruct((N, 1), jnp.float32),
        ),
    )(wp, sp, hv, W_et, b_et.reshape(1, H),
      Wih1, bih1.reshape(1, 3 * H), Whh1, bhh1.reshape(1, 3 * H),
      wla.reshape(H, 1), wlb.reshape(H, 1))


def _edge2_sc(qatab, qbtab, src3d, dst3d, bvec, z1):
    @functools.partial(
        pl.kernel, mesh=_mesh(), compiler_params=_SC_PARAMS,
        out_type=(
            jax.ShapeDtypeStruct((NG, RST, RW), jnp.float32),
            jax.ShapeDtypeStruct((2, NP), jnp.float32),
        ),
        scratch_types=[
            pltpu.VMEM((NP,), jnp.float32),
            pltpu.VMEM((NP,), jnp.float32),
            pltpu.VMEM((16,), jnp.float32),
            pltpu.VMEM((RST, RW), jnp.int32),
            pltpu.VMEM((RST, RW), jnp.int32),
            pltpu.VMEM((RST, RW), jnp.float32),
            pltpu.VMEM_SHARED((NP,), jnp.float32),
        ])
    def k(qa_h, qb_h, s_h, d_h, b_h, z1_h, e2_h, sp_h,
          qat, qbt, bbuf, sbuf, dbuf, obuf, ssh):
        cid = lax.axis_index("c")
        sid = lax.axis_index("s")
        wid = sid * 2 + cid
        nrow = NP // 16
        pltpu.sync_copy(qa_h, qat)
        pltpu.sync_copy(qb_h, qbt)
        pltpu.sync_copy(b_h, bbuf)
        pltpu.sync_copy(z1_h, ssh.at[pl.ds(sid * nrow, nrow)])
        plsc.subcore_barrier()
        bv = bbuf[...]

        def stage(t, _):
            gidx = wid * GPW + t
            pltpu.sync_copy(s_h.at[gidx], sbuf)
            pltpu.sync_copy(d_h.at[gidx], dbuf)

            def inner(j, _):
                for kk in range(RW // 16):
                    iv = dbuf[j, pl.ds(kk * 16, 16)]
                    iv2 = sbuf[j, pl.ds(kk * 16, 16)]
                    lv = (plsc.load_gather(qat, [iv])
                          + plsc.load_gather(qbt, [iv2]) + bv)
                    lv = jnp.where(lv >= 0, lv, 0.01 * lv)
                    obuf[j, pl.ds(kk * 16, 16)] = jnp.exp(lv)
                pltpu.sync_copy(obuf.at[j], ssh.at[dbuf.at[j]], add=True)
                return 0

            lax.fori_loop(0, RST, inner, 0)
            pltpu.sync_copy(obuf, e2_h.at[gidx])
            return 0

        lax.fori_loop(0, GPW, stage, 0)
        plsc.subcore_barrier()
        pltpu.sync_copy(ssh.at[pl.ds(sid * nrow, nrow)],
                        sp_h.at[cid, pl.ds(sid * nrow, nrow)])

    return k(qatab, qbtab, src3d, dst3d, bvec, z1)


def _gather_scale_scatter(table, src3d, e23d, dst3d, zrows):
    @functools.partial(
        pl.kernel, mesh=_mesh(), compiler_params=_SC_PARAMS,
        out_type=jax.ShapeDtypeStruct((2, NP, H), jnp.float32),
        scratch_types=[
            pltpu.VMEM((RST, RW), jnp.int32),
            pltpu.VMEM((RST, RW), jnp.int32),
            pltpu.VMEM((RST, RW), jnp.float32),
            pltpu.VMEM((RW, H), jnp.float32),
            pltpu.VMEM_SHARED((NP, H), jnp.float32),
            pltpu.SemaphoreType.DMA,
        ])
    def k(tab_h, s_h, e_h, d_h, zr_h, tp_h, sbuf, dbuf, ebuf, rbuf, csh, sem):
        cid = lax.axis_index("c")
        sid = lax.axis_index("s")
        wid = sid * 2 + cid
        nrow = NP // 16
        pltpu.sync_copy(zr_h, csh.at[pl.ds(sid * nrow, nrow)])
        plsc.subcore_barrier()

        def stage(t, _):
            gidx = wid * GPW + t
            pltpu.sync_copy(s_h.at[gidx], sbuf)
            pltpu.sync_copy(d_h.at[gidx], dbuf)
            pltpu.sync_copy(e_h.at[gidx], ebuf)

            def inner(j, _):
                pltpu.async_copy(tab_h.at[sbuf.at[j]], rbuf, sem).wait()
                for kb in range(RW // 16):
                    av16 = ebuf[j, pl.ds(kb * 16, 16)]
                    for rr in range(16):
                        av = jnp.full((16,), av16[rr], jnp.float32)
                        row = kb * 16 + rr
                        for cc in range(H // 16):
                            rbuf[row, pl.ds(cc * 16, 16)] = (
                                rbuf[row, pl.ds(cc * 16, 16)] * av)
                pltpu.sync_copy(rbuf, csh.at[dbuf.at[j]], add=True)
                return 0

            lax.fori_loop(0, RST, inner, 0)
            return 0

        lax.fori_loop(0, GPW, stage, 0)
        plsc.subcore_barrier()
        pltpu.sync_copy(csh.at[pl.ds(sid * nrow, nrow)],
                        tp_h.at[cid, pl.ds(sid * nrow, nrow)])

    return k(table, src3d, e23d, dst3d, zrows)


def _gru2_pool(tp, s2p, h1, W_pn2, b_pn2, Wih2, bih2, Whh2, bhh2, gids):
    NBN = 10
    BN = N // NBN

    def body(tp_r, sp_r, h1_r, wpn_r, bpn_r, wih_r, bih_r, whh_r, bhh_r,
             gid_r, out_r, cnt_r):
        i = pl.program_id(0)

        @pl.when(i == 0)
        def _():
            out_r[...] = jnp.zeros_like(out_r)
            cnt_r[...] = jnp.zeros_like(cnt_r)

        c = _ctx(tp_r[...], sp_r[...], wpn_r[...], bpn_r[...])
        h2 = jax.nn.relu(_gru_block(_elu(c), h1_r[...], wih_r[...], bih_r[...],
                                    whh_r[...], bhh_r[...]))
        onehot = (gid_r[...] == lax.broadcasted_iota(jnp.int32, (BN, G), 1)
                  ).astype(jnp.float32)
        out_r[...] += lax.dot_general(onehot, h2, (((0,), (0,)), ((), ())),
                                      preferred_element_type=jnp.float32)
        cnt_r[...] += lax.dot_general(
            onehot, jnp.ones((BN, 1), jnp.float32),
            (((0,), (0,)), ((), ())), preferred_element_type=jnp.float32)

        @pl.when(i == NBN - 1)
        def _():
            out_r[...] = out_r[...] / jnp.maximum(cnt_r[...], 1.0)

    full2 = lambda shape: pl.BlockSpec(shape, lambda i: (0, 0))
    return pl.pallas_call(
        body,
        grid=(NBN,),
        in_specs=[
            pl.BlockSpec((2, BN, H), lambda i: (0, i, 0)),
            pl.BlockSpec((2, BN, 1), lambda i: (0, i, 0)),
            pl.BlockSpec((BN, H), lambda i: (i, 0)),
            full2((H, H)), full2((1, H)),
            full2((3 * H, H)), full2((1, 3 * H)),
            full2((3 * H, H)), full2((1, 3 * H)),
            pl.BlockSpec((BN, 1), lambda i: (i, 0)),
        ],
        out_specs=pl.BlockSpec((G, H), lambda i: (0, 0)),
        out_shape=jax.ShapeDtypeStruct((G, H), jnp.float32),
        scratch_shapes=[pltpu.VMEM((G, 1), jnp.float32)],
    )(tp, s2p, h1, W_pn2, b_pn2.reshape(1, H),
      Wih2, bih2.reshape(1, 3 * H), Whh2, bhh2.reshape(1, 3 * H),
      gids.reshape(N, 1))


def kernel(x, edge_attr, edge_index, graph_ids,
           W_pn, b_pn, W_pe1, b_pe1, W_pe2, b_pe2, W_et, b_et,
           Wih1, bih1, Whh1, bhh1,
           W_pe_l, b_pe_l, W_pn2, b_pn2, Wih2, bih2, Whh2, bhh2):
    src3d = edge_index[0].reshape(NG, RST, RW)
    dst3d = edge_index[1].reshape(NG, RST, RW)

    W_pe1x = W_pe1[:, :FN]
    W_pe1e = W_pe1[:, FN:]
    w2a = W_pe2[0, :H]
    w2b = W_pe2[0, H:]
    wla = W_pe_l[0, :H]
    wlb = W_pe_l[0, H:]

    z1 = jnp.zeros((NP // 16,), jnp.float32)
    zrows = jnp.zeros((NP // 16, H), jnp.float32)

    def padN(v):
        return jnp.concatenate([v.reshape(N), jnp.zeros((NP - N,), jnp.float32)])

    hv, xp, q = _node_prep(x, W_pn, b_pn, W_pe1x, w2a)
    xs, qd3d = _gather1(xp, src3d, dst3d, padN(q))
    he1e, el = _edge1(xs, edge_attr, qd3d.reshape(E, 1),
                      W_pe1e, b_pe1, w2b, b_pe2)
    wp, sp = _scatter_l1(he1e, el.reshape(NG, RST, RW), dst3d, zrows, z1)
    h1, qa, qb = _gru1(wp, sp.reshape(2, NP, 1), hv, W_et, b_et,
                       Wih1, bih1, Whh1, bhh1, wla, wlb)

    bl = jnp.full((16,), b_pe_l[0], jnp.float32)
    e23d, s2p = _edge2_sc(padN(qa), padN(qb), src3d, dst3d, bl, z1)
    tp = _gather_scale_scatter(h1, src3d, e23d, dst3d, zrows)
    return _gru2_pool(tp, s2p.reshape(2, NP, 1), h1, W_pn2, b_pn2,
                      Wih2, bih2, Whh2, bhh2, graph_ids)


# speedup vs baseline: 10.8932x; 1.1412x over previous
import functools

import jax
import jax.numpy as jnp
from jax import lax
from jax.experimental import pallas as pl
from jax.experimental.pallas import tpu as pltpu
from jax.experimental.pallas import tpu_sc as plsc

N = 10000
E = 320000
G = 256
H = 128
FN = 74
FE = 12

NP = 10240
RW = 80
RST = 25
NG = E // (RW * RST)
NWORK = 32
GPW = NG // NWORK
BE = 2560
NB = E // BE

_SC_PARAMS = pltpu.CompilerParams(needs_layout_passes=False)


def _leaky(v):
    return jnp.where(v >= 0, v, 0.01 * v)


def _mesh():
    return plsc.VectorSubcoreMesh(core_axis_name="c", subcore_axis_name="s")


def _node_prep(x, W_pn, b_pn, W_pe1x, w2a):
    def body(x_r, wpn_r, bpn_r, wpx_r, w2a_r, hv_r, xp_r, q_r):
        xv = x_r[...]
        hv = _leaky(lax.dot_general(xv, wpn_r[...], (((1,), (1,)), ((), ())),
                                    preferred_element_type=jnp.float32)
                    + bpn_r[...])
        hv_r[...] = hv
        xp_r[...] = lax.dot_general(xv, wpx_r[...], (((1,), (1,)), ((), ())),
                                    preferred_element_type=jnp.float32)
        q_r[...] = lax.dot_general(hv, w2a_r[...], (((1,), (0,)), ((), ())),
                                   preferred_element_type=jnp.float32)

    return pl.pallas_call(
        body,
        out_shape=(
            jax.ShapeDtypeStruct((N, H), jnp.float32),
            jax.ShapeDtypeStruct((N, H), jnp.float32),
            jax.ShapeDtypeStruct((N, 1), jnp.float32),
        ),
    )(x, W_pn, b_pn.reshape(1, H), W_pe1x, w2a.reshape(H, 1))


def _gather1(table, src3d, dst3d, qtab):
    @functools.partial(
        pl.kernel, mesh=_mesh(), compiler_params=_SC_PARAMS,
        out_type=(
            jax.ShapeDtypeStruct((E, H), jnp.float32),
            jax.ShapeDtypeStruct((NG, RST, RW), jnp.float32),
        ),
        scratch_types=[
            pltpu.VMEM((NP,), jnp.float32),
            pltpu.VMEM((RST, RW), jnp.int32),
            pltpu.VMEM((RST, RW), jnp.int32),
            pltpu.VMEM((RW, H), jnp.float32),
            pltpu.VMEM((RW, H), jnp.float32),
            pltpu.VMEM((RST, RW), jnp.float32),
            pltpu.SemaphoreType.DMA,
            pltpu.SemaphoreType.DMA,
        ])
    def k(tab_h, s_h, d_h, q_h, rows_h, sc_h,
          qt, sbuf, dbuf, rbufa, rbufb, obuf, sema, semb):
        cid = lax.axis_index("c")
        sid = lax.axis_index("s")
        wid = sid * 2 + cid
        pltpu.sync_copy(q_h, qt)

        def qgather(j):
            for kk in range(RW // 16):
                iv = dbuf[j, pl.ds(kk * 16, 16)]
                obuf[j, pl.ds(kk * 16, 16)] = plsc.load_gather(qt, [iv])

        def stage(t, _):
            gidx = wid * GPW + t
            pltpu.sync_copy(s_h.at[gidx], sbuf)
            pltpu.sync_copy(d_h.at[gidx], dbuf)
            e0 = gidx * RST * RW
            pltpu.async_copy(tab_h.at[sbuf.at[0]], rbufa, sema)

            def dbl(tt, _):
                ja = 2 * tt
                jb = 2 * tt + 1
                pltpu.make_async_copy(
                    tab_h.at[pl.ds(0, RW)], rbufa, sema).wait()
                pltpu.async_copy(tab_h.at[sbuf.at[jb]], rbufb, semb)
                qgather(ja)
                pltpu.sync_copy(rbufa, rows_h.at[pl.ds(e0 + ja * RW, RW)])
                pltpu.make_async_copy(
                    tab_h.at[pl.ds(0, RW)], rbufb, semb).wait()
                pltpu.async_copy(tab_h.at[sbuf.at[jb + 1]], rbufa, sema)
                qgather(jb)
                pltpu.sync_copy(rbufb, rows_h.at[pl.ds(e0 + jb * RW, RW)])
                return 0

            lax.fori_loop(0, (RST - 1) // 2, dbl, 0)
            pltpu.make_async_copy(tab_h.at[pl.ds(0, RW)], rbufa, sema).wait()
            qgather(RST - 1)
            pltpu.sync_copy(rbufa, rows_h.at[pl.ds(e0 + (RST - 1) * RW, RW)])
            pltpu.sync_copy(obuf, sc_h.at[gidx])
            return 0

        lax.fori_loop(0, GPW, stage, 0)

    return k(table, src3d, dst3d, qtab)


def _edge1(xs, edge_attr, qd, W_pe1e, b_pe1, w2b, b_pe2):
    def body(xs_r, ea_r, qd_r, wpe_r, bp1_r, w2b_r, bp2_r, he_r, el_r):
        ep = lax.dot_general(ea_r[...], wpe_r[...], (((1,), (1,)), ((), ())),
                             preferred_element_type=jnp.float32)
        he1 = _leaky(xs_r[...] + ep + bp1_r[...])
        r = lax.dot_general(he1, w2b_r[...], (((1,), (0,)), ((), ())),
                            preferred_element_type=jnp.float32)
        e = jnp.exp(_leaky(qd_r[...] + r + bp2_r[...]))
        el_r[...] = e
        he_r[...] = he1 * e

    full = lambda shape: pl.BlockSpec(shape, lambda i: (0, 0))
    return pl.pallas_call(
        body,
        grid=(NB,),
        in_specs=[
            pl.BlockSpec((BE, H), lambda i: (i, 0)),
            pl.BlockSpec((BE, FE), lambda i: (i, 0)),
            pl.BlockSpec((BE, 1), lambda i: (i, 0)),
            full((H, FE)), full((1, H)), full((H, 1)), full((1, 1)),
        ],
        out_specs=[
            pl.BlockSpec((BE, H), lambda i: (i, 0)),
            pl.BlockSpec((BE, 1), lambda i: (i, 0)),
        ],
        out_shape=(
            jax.ShapeDtypeStruct((E, H), jnp.float32),
            jax.ShapeDtypeStruct((E, 1), jnp.float32),
        ),
    )(xs, edge_attr, qd, W_pe1e, b_pe1.reshape(1, H), w2b.reshape(H, 1),
      b_pe2.reshape(1, 1))


def _scatter_l1(rows, el3d, dst3d, zrows, z1):
    @functools.partial(
        pl.kernel, mesh=_mesh(), compiler_params=_SC_PARAMS,
        out_type=(
            jax.ShapeDtypeStruct((2, NP, H), jnp.float32),
            jax.ShapeDtypeStruct((2, NP), jnp.float32),
        ),
        scratch_types=[
            pltpu.VMEM((RW, H), jnp.float32),
            pltpu.VMEM((RW, H), jnp.float32),
            pltpu.VMEM((RST, RW), jnp.int32),
            pltpu.VMEM((RST, RW), jnp.float32),
            pltpu.VMEM_SHARED((NP, H), jnp.float32),
            pltpu.VMEM_SHARED((NP,), jnp.float32),
            pltpu.SemaphoreType.DMA,
            pltpu.SemaphoreType.DMA,
        ])
    def k(m_h, e_h, d_h, zr_h, z1_h, cp_h, sp_h,
          mbufa, mbufb, dbuf, ebuf, csh, ssh, sema, semb):
        cid = lax.axis_index("c")
        sid = lax.axis_index("s")
        wid = sid * 2 + cid
        nrow = NP // 16
        pltpu.sync_copy(zr_h, csh.at[pl.ds(sid * nrow, nrow)])
        pltpu.sync_copy(z1_h, ssh.at[pl.ds(sid * nrow, nrow)])
        plsc.subcore_barrier()

        def stage(t, _):
            gidx = wid * GPW + t
            pltpu.sync_copy(d_h.at[gidx], dbuf)
            pltpu.sync_copy(e_h.at[gidx], ebuf)
            e0 = gidx * RST * RW
            pltpu.async_copy(m_h.at[pl.ds(e0, RW)], mbufa, sema)

            def scat(buf, j):
                pltpu.sync_copy(buf, csh.at[dbuf.at[j]], add=True)
                pltpu.sync_copy(ebuf.at[j], ssh.at[dbuf.at[j]], add=True)

            def dbl(tt, _):
                ja = 2 * tt
                jb = 2 * tt + 1
                pltpu.make_async_copy(
                    m_h.at[pl.ds(0, RW)], mbufa, sema).wait()
                pltpu.async_copy(
                    m_h.at[pl.ds(e0 + jb * RW, RW)], mbufb, semb)
                scat(mbufa, ja)
                pltpu.make_async_copy(
                    m_h.at[pl.ds(0, RW)], mbufb, semb).wait()
                pltpu.async_copy(
                    m_h.at[pl.ds(e0 + (jb + 1) * RW, RW)], mbufa, sema)
                scat(mbufb, jb)
                return 0

            lax.fori_loop(0, (RST - 1) // 2, dbl, 0)
            pltpu.make_async_copy(m_h.at[pl.ds(0, RW)], mbufa, sema).wait()
            scat(mbufa, RST - 1)
            return 0

        lax.fori_loop(0, GPW, stage, 0)
        plsc.subcore_barrier()
        pltpu.sync_copy(csh.at[pl.ds(sid * nrow, nrow)],
                        cp_h.at[cid, pl.ds(sid * nrow, nrow)])
        pltpu.sync_copy(ssh.at[pl.ds(sid * nrow, nrow)],
                        sp_h.at[cid, pl.ds(sid * nrow, nrow)])

    return k(rows, el3d, dst3d, zrows, z1)


def _gru_block(xg, h, Wih, bih, Whh, bhh):
    gi = lax.dot_general(xg, Wih, (((1,), (1,)), ((), ())),
                         preferred_element_type=jnp.float32) + bih
    gh = lax.dot_general(h, Whh, (((1,), (1,)), ((), ())),
                         preferred_element_type=jnp.float32) + bhh
    i_r, i_z, i_n = gi[:, :H], gi[:, H:2 * H], gi[:, 2 * H:]
    h_r, h_z, h_n = gh[:, :H], gh[:, H:2 * H], gh[:, 2 * H:]
    r = jax.nn.sigmoid(i_r + h_r)
    z = jax.nn.sigmoid(i_z + h_z)
    n = jnp.tanh(i_n + r * h_n)
    return (1.0 - z) * n + z * h


def _elu(v):
    return jnp.where(v > 0, v, jnp.exp(v) - 1.0)


def _ctx(wp_r, sp_r, wmat_r, bias_r):
    s = sp_r[0] + sp_r[1]
    inv = jnp.where(s > 0, 1.0 / jnp.maximum(s, 1e-30), 0.0)
    msk = (s > 0).astype(jnp.float32)
    w = (wp_r[0] + wp_r[1]) * inv
    return lax.dot_general(w, wmat_r, (((1,), (1,)), ((), ())),
                           preferred_element_type=jnp.float32) + msk * bias_r


def _gru1(wp, sp, hv, W_et, b_et, Wih1, bih1, Whh1, bhh1, wla, wlb):
    NBN = 5
    BN = N // NBN

    def body(wp_r, sp_r, hv_r, wet_r, bet_r, wih_r, bih_r, whh_r, bhh_r,
             wla_r, wlb_r, h1_r, qa_r, qb_r):
        c = _ctx(wp_r[...], sp_r[...], wet_r[...], bet_r[...])
        h1 = jax.nn.relu(_gru_block(_elu(c), hv_r[...], wih_r[...], bih_r[...],
                                    whh_r[...], bhh_r[...]))
        h1_r[...] = h1
        qa_r[...] = lax.dot_general(h1, wla_r[...], (((1,), (0,)), ((), ())),
                                    preferred_element_type=jnp.float32)
        qb_r[...] = lax.dot_general(h1, wlb_r[...], (((1,), (0,)), ((), ())),
                                    preferred_element_type=jnp.float32)

    full2 = lambda shape: pl.BlockSpec(shape, lambda i: (0, 0))
    return pl.pallas_call(
        body,
        grid=(NBN,),
        in_specs=[
            pl.BlockSpec((2, BN, H), lambda i: (0, i, 0)),
            pl.BlockSpec((2, BN, 1), lambda i: (0, i, 0)),
            pl.BlockSpec((BN, H), lambda i: (i, 0)),
            full2((H, H)), full2((1, H)),
            full2((3 * H, H)), full2((1, 3 * H)),
            full2((3 * H, H)), full2((1, 3 * H)),
            full2((H, 1)), full2((H, 1)),
        ],
        out_specs=[
            pl.BlockSpec((BN, H), lambda i: (i, 0)),
            pl.BlockSpec((BN, 1), lambda i: (i, 0)),
            pl.BlockSpec((BN, 1), lambda i: (i, 0)),
        ],
        out_shape=(
            jax.ShapeDtypeStruct((N, H), jnp.float32),
            jax.ShapeDtypeStruct((N, 1), jnp.float32),
            jax.ShapeDtypeStruct((N, 1), jnp.float32),
        ),
    )(wp, sp, hv, W_et, b_et.reshape(1, H),
      Wih1, bih1.reshape(1, 3 * H), Whh1, bhh1.reshape(1, 3 * H),
      wla.reshape(H, 1), wlb.reshape(H, 1))


def _edge2_sc(qatab, qbtab, src3d, dst3d, bvec, z1):
    @functools.partial(
        pl.kernel, mesh=_mesh(), compiler_params=_SC_PARAMS,
        out_type=(
            jax.ShapeDtypeStruct((NG, RST, RW), jnp.float32),
            jax.ShapeDtypeStruct((2, NP), jnp.float32),
        ),
        scratch_types=[
            pltpu.VMEM((NP,), jnp.float32),
            pltpu.VMEM((NP,), jnp.float32),
            pltpu.VMEM((16,), jnp.float32),
            pltpu.VMEM((RST, RW), jnp.int32),
            pltpu.VMEM((RST, RW), jnp.int32),
            pltpu.VMEM((RST, RW), jnp.float32),
            pltpu.VMEM_SHARED((NP,), jnp.float32),
        ])
    def k(qa_h, qb_h, s_h, d_h, b_h, z1_h, e2_h, sp_h,
          qat, qbt, bbuf, sbuf, dbuf, obuf, ssh):
        cid = lax.axis_index("c")
        sid = lax.axis_index("s")
        wid = sid * 2 + cid
        nrow = NP // 16
        pltpu.sync_copy(qa_h, qat)
        pltpu.sync_copy(qb_h, qbt)
        pltpu.sync_copy(b_h, bbuf)
        pltpu.sync_copy(z1_h, ssh.at[pl.ds(sid * nrow, nrow)])
        plsc.subcore_barrier()
        bv = bbuf[...]

        def stage(t, _):
            gidx = wid * GPW + t
            pltpu.sync_copy(s_h.at[gidx], sbuf)
            pltpu.sync_copy(d_h.at[gidx], dbuf)

            def inner(j, _):
                for kk in range(RW // 16):
                    iv = dbuf[j, pl.ds(kk * 16, 16)]
                    iv2 = sbuf[j, pl.ds(kk * 16, 16)]
                    lv = (plsc.load_gather(qat, [iv])
                          + plsc.load_gather(qbt, [iv2]) + bv)
                    lv = jnp.where(lv >= 0, lv, 0.01 * lv)
                    obuf[j, pl.ds(kk * 16, 16)] = jnp.exp(lv)
                pltpu.sync_copy(obuf.at[j], ssh.at[dbuf.at[j]], add=True)
                return 0

            lax.fori_loop(0, RST, inner, 0)
            pltpu.sync_copy(obuf, e2_h.at[gidx])
            return 0

        lax.fori_loop(0, GPW, stage, 0)
        plsc.subcore_barrier()
        pltpu.sync_copy(ssh.at[pl.ds(sid * nrow, nrow)],
                        sp_h.at[cid, pl.ds(sid * nrow, nrow)])

    return k(qatab, qbtab, src3d, dst3d, bvec, z1)


def _gather_scale_scatter(table, src3d, e23d, dst3d, zrows):
    @functools.partial(
        pl.kernel, mesh=_mesh(), compiler_params=_SC_PARAMS,
        out_type=jax.ShapeDtypeStruct((2, NP, H), jnp.float32),
        scratch_types=[
            pltpu.VMEM((RST, RW), jnp.int32),
            pltpu.VMEM((RST, RW), jnp.int32),
            pltpu.VMEM((RST, RW), jnp.float32),
            pltpu.VMEM((RW, H), jnp.float32),
            pltpu.VMEM((RW, H), jnp.float32),
            pltpu.VMEM_SHARED((NP, H), jnp.float32),
            pltpu.SemaphoreType.DMA,
            pltpu.SemaphoreType.DMA,
        ])
    def k(tab_h, s_h, e_h, d_h, zr_h, tp_h,
          sbuf, dbuf, ebuf, rbufa, rbufb, csh, sema, semb):
        cid = lax.axis_index("c")
        sid = lax.axis_index("s")
        wid = sid * 2 + cid
        nrow = NP // 16
        pltpu.sync_copy(zr_h, csh.at[pl.ds(sid * nrow, nrow)])
        plsc.subcore_barrier()

        def stage(t, _):
            gidx = wid * GPW + t
            pltpu.sync_copy(s_h.at[gidx], sbuf)
            pltpu.sync_copy(d_h.at[gidx], dbuf)
            pltpu.sync_copy(e_h.at[gidx], ebuf)
            pltpu.async_copy(tab_h.at[sbuf.at[0]], rbufa, sema)

            def scale_scat(buf, j):
                for kb in range(RW // 16):
                    av16 = ebuf[j, pl.ds(kb * 16, 16)]
                    for rr in range(16):
                        av = jnp.full((16,), av16[rr], jnp.float32)
                        row = kb * 16 + rr
                        for cc in range(H // 16):
                            buf[row, pl.ds(cc * 16, 16)] = (
                                buf[row, pl.ds(cc * 16, 16)] * av)
                pltpu.sync_copy(buf, csh.at[dbuf.at[j]], add=True)

            def dbl(tt, _):
                ja = 2 * tt
                jb = 2 * tt + 1
                pltpu.make_async_copy(
                    tab_h.at[pl.ds(0, RW)], rbufa, sema).wait()
                pltpu.async_copy(tab_h.at[sbuf.at[jb]], rbufb, semb)
                scale_scat(rbufa, ja)
                pltpu.make_async_copy(
                    tab_h.at[pl.ds(0, RW)], rbufb, semb).wait()
                pltpu.async_copy(tab_h.at[sbuf.at[jb + 1]], rbufa, sema)
                scale_scat(rbufb, jb)
                return 0

            lax.fori_loop(0, (RST - 1) // 2, dbl, 0)
            pltpu.make_async_copy(tab_h.at[pl.ds(0, RW)], rbufa, sema).wait()
            scale_scat(rbufa, RST - 1)
            return 0

        lax.fori_loop(0, GPW, stage, 0)
        plsc.subcore_barrier()
        pltpu.sync_copy(csh.at[pl.ds(sid * nrow, nrow)],
                        tp_h.at[cid, pl.ds(sid * nrow, nrow)])

    return k(table, src3d, e23d, dst3d, zrows)


def _gru2_pool(tp, s2p, h1, W_pn2, b_pn2, Wih2, bih2, Whh2, bhh2, gids):
    NBN = 10
    BN = N // NBN

    def body(tp_r, sp_r, h1_r, wpn_r, bpn_r, wih_r, bih_r, whh_r, bhh_r,
             gid_r, out_r, cnt_r):
        i = pl.program_id(0)

        @pl.when(i == 0)
        def _():
            out_r[...] = jnp.zeros_like(out_r)
            cnt_r[...] = jnp.zeros_like(cnt_r)

        c = _ctx(tp_r[...], sp_r[...], wpn_r[...], bpn_r[...])
        h2 = jax.nn.relu(_gru_block(_elu(c), h1_r[...], wih_r[...], bih_r[...],
                                    whh_r[...], bhh_r[...]))
        onehot = (gid_r[...] == lax.broadcasted_iota(jnp.int32, (BN, G), 1)
                  ).astype(jnp.float32)
        out_r[...] += lax.dot_general(onehot, h2, (((0,), (0,)), ((), ())),
                                      preferred_element_type=jnp.float32)
        cnt_r[...] += lax.dot_general(
            onehot, jnp.ones((BN, 1), jnp.float32),
            (((0,), (0,)), ((), ())), preferred_element_type=jnp.float32)

        @pl.when(i == NBN - 1)
        def _():
            out_r[...] = out_r[...] / jnp.maximum(cnt_r[...], 1.0)

    full2 = lambda shape: pl.BlockSpec(shape, lambda i: (0, 0))
    return pl.pallas_call(
        body,
        grid=(NBN,),
        in_specs=[
            pl.BlockSpec((2, BN, H), lambda i: (0, i, 0)),
            pl.BlockSpec((2, BN, 1), lambda i: (0, i, 0)),
            pl.BlockSpec((BN, H), lambda i: (i, 0)),
            full2((H, H)), full2((1, H)),
            full2((3 * H, H)), full2((1, 3 * H)),
            full2((3 * H, H)), full2((1, 3 * H)),
            pl.BlockSpec((BN, 1), lambda i: (i, 0)),
        ],
        out_specs=pl.BlockSpec((G, H), lambda i: (0, 0)),
        out_shape=jax.ShapeDtypeStruct((G, H), jnp.float32),
        scratch_shapes=[pltpu.VMEM((G, 1), jnp.float32)],
    )(tp, s2p, h1, W_pn2, b_pn2.reshape(1, H),
      Wih2, bih2.reshape(1, 3 * H), Whh2, bhh2.reshape(1, 3 * H),
      gids.reshape(N, 1))


def kernel(x, edge_attr, edge_index, graph_ids,
           W_pn, b_pn, W_pe1, b_pe1, W_pe2, b_pe2, W_et, b_et,
           Wih1, bih1, Whh1, bhh1,
           W_pe_l, b_pe_l, W_pn2, b_pn2, Wih2, bih2, Whh2, bhh2):
    src3d = edge_index[0].reshape(NG, RST, RW)
    dst3d = edge_index[1].reshape(NG, RST, RW)

    W_pe1x = W_pe1[:, :FN]
    W_pe1e = W_pe1[:, FN:]
    w2a = W_pe2[0, :H]
    w2b = W_pe2[0, H:]
    wla = W_pe_l[0, :H]
    wlb = W_pe_l[0, H:]

    z1 = jnp.zeros((NP // 16,), jnp.float32)
    zrows = jnp.zeros((NP // 16, H), jnp.float32)

    def padN(v):
        return jnp.concatenate([v.reshape(N), jnp.zeros((NP - N,), jnp.float32)])

    hv, xp, q = _node_prep(x, W_pn, b_pn, W_pe1x, w2a)
    xs, qd3d = _gather1(xp, src3d, dst3d, padN(q))
    he1e, el = _edge1(xs, edge_attr, qd3d.reshape(E, 1),
                      W_pe1e, b_pe1, w2b, b_pe2)
    wp, sp = _scatter_l1(he1e, el.reshape(NG, RST, RW), dst3d, zrows, z1)
    h1, qa, qb = _gru1(wp, sp.reshape(2, NP, 1), hv, W_et, b_et,
                       Wih1, bih1, Whh1, bhh1, wla, wlb)

    bl = jnp.full((16,), b_pe_l[0], jnp.float32)
    e23d, s2p = _edge2_sc(padN(qa), padN(qb), src3d, dst3d, bl, z1)
    tp = _gather_scale_scatter(h1, src3d, e23d, dst3d, zrows)
    return _gru2_pool(tp, s2p.reshape(2, NP, 1), h1, W_pn2, b_pn2,
                      Wih2, bih2, Whh2, bhh2, graph_ids)


# speedup vs baseline: 15.3933x; 1.4131x over previous
import functools

import jax
import jax.numpy as jnp
from jax import lax
from jax.experimental import pallas as pl
from jax.experimental.pallas import tpu as pltpu
from jax.experimental.pallas import tpu_sc as plsc

N = 10000
E = 320000
G = 256
H = 128
FN = 74
FE = 12

NP = 10240
RW = 80
RST = 25
NG = E // (RW * RST)
NWORK = 32
GPW = NG // NWORK
BE = 2560
NB = E // BE

_SC_PARAMS = pltpu.CompilerParams(needs_layout_passes=False)


def _leaky(v):
    return jnp.where(v >= 0, v, 0.01 * v)


def _mesh():
    return plsc.VectorSubcoreMesh(core_axis_name="c", subcore_axis_name="s")


def _node_prep(x, W_pn, b_pn, W_pe1x, w2a):
    def body(x_r, wpn_r, bpn_r, wpx_r, w2a_r, hv_r, xp_r, q_r):
        xv = x_r[...]
        hv = _leaky(lax.dot_general(xv, wpn_r[...], (((1,), (1,)), ((), ())),
                                    preferred_element_type=jnp.float32)
                    + bpn_r[...])
        hv_r[...] = hv
        xp_r[...] = lax.dot_general(xv, wpx_r[...], (((1,), (1,)), ((), ())),
                                    preferred_element_type=jnp.float32)
        q_r[pl.ds(0, N), :] = lax.dot_general(
            hv, w2a_r[...], (((1,), (0,)), ((), ())),
            preferred_element_type=jnp.float32)

    return pl.pallas_call(
        body,
        out_shape=(
            jax.ShapeDtypeStruct((N, H), jnp.float32),
            jax.ShapeDtypeStruct((N, H), jnp.float32),
            jax.ShapeDtypeStruct((NP, 1), jnp.float32),
        ),
    )(x, W_pn, b_pn.reshape(1, H), W_pe1x, w2a.reshape(H, 1))


def _gather1(table, src3d, dst3d, qtab, ng):
    gpw = ng // NWORK
    ne = ng * RST * RW
    @functools.partial(
        pl.kernel, mesh=_mesh(), compiler_params=_SC_PARAMS,
        out_type=(
            jax.ShapeDtypeStruct((ne, H), jnp.float32),
            jax.ShapeDtypeStruct((ne,), jnp.float32),
        ),
        scratch_types=[
            pltpu.VMEM((NP,), jnp.float32),
            pltpu.VMEM((RST, RW), jnp.int32),
            pltpu.VMEM((RST, RW), jnp.int32),
            pltpu.VMEM((RW, H), jnp.float32),
            pltpu.VMEM((RW, H), jnp.float32),
            pltpu.VMEM((RST * RW,), jnp.float32),
            pltpu.SemaphoreType.DMA,
            pltpu.SemaphoreType.DMA,
        ])
    def k(tab_h, s_h, d_h, q_h, rows_h, sc_h,
          qt, sbuf, dbuf, rbufa, rbufb, obuf, sema, semb):
        cid = lax.axis_index("c")
        sid = lax.axis_index("s")
        wid = sid * 2 + cid
        pltpu.sync_copy(q_h, qt)

        def qgather(j):
            for kk in range(RW // 16):
                iv = dbuf[j, pl.ds(kk * 16, 16)]
                obuf[pl.ds(j * RW + kk * 16, 16)] = plsc.load_gather(qt, [iv])

        def stage(t, _):
            gidx = wid * gpw + t
            pltpu.sync_copy(s_h.at[gidx], sbuf)
            pltpu.sync_copy(d_h.at[gidx], dbuf)
            e0 = gidx * RST * RW
            pltpu.async_copy(tab_h.at[sbuf.at[0]], rbufa, sema)

            def dbl(tt, _):
                ja = 2 * tt
                jb = 2 * tt + 1
                pltpu.make_async_copy(
                    tab_h.at[pl.ds(0, RW)], rbufa, sema).wait()
                pltpu.async_copy(tab_h.at[sbuf.at[jb]], rbufb, semb)
                qgather(ja)
                pltpu.sync_copy(rbufa, rows_h.at[pl.ds(e0 + ja * RW, RW)])
                pltpu.make_async_copy(
                    tab_h.at[pl.ds(0, RW)], rbufb, semb).wait()
                pltpu.async_copy(tab_h.at[sbuf.at[jb + 1]], rbufa, sema)
                qgather(jb)
                pltpu.sync_copy(rbufb, rows_h.at[pl.ds(e0 + jb * RW, RW)])
                return 0

            lax.fori_loop(0, (RST - 1) // 2, dbl, 0)
            pltpu.make_async_copy(tab_h.at[pl.ds(0, RW)], rbufa, sema).wait()
            qgather(RST - 1)
            pltpu.sync_copy(rbufa, rows_h.at[pl.ds(e0 + (RST - 1) * RW, RW)])
            pltpu.sync_copy(obuf, sc_h.at[pl.ds(e0, RST * RW)])
            return 0

        lax.fori_loop(0, gpw, stage, 0)

    return k(table, src3d, dst3d, qtab)


def _edge1(xs, eat, qd128, W_pe1e, b_pe1, w2b, b_pe2, nb, base):
    BR = BE // 128
    ne = nb * BE

    def body(xs_r, ea_r, qd_r, wpe_r, bp1_r, w2b_r, bp2_r, he_r, el_r):
        ep = lax.dot_general(ea_r[...], wpe_r[...], (((0,), (1,)), ((), ())),
                             preferred_element_type=jnp.float32)
        he1 = _leaky(xs_r[...] + ep + bp1_r[...])
        r = lax.dot_general(he1, w2b_r[...], (((1,), (0,)), ((), ())),
                            preferred_element_type=jnp.float32)
        e = jnp.exp(_leaky(qd_r[0] + r.reshape(BR, 128) + bp2_r[...]))
        el_r[...] = e.reshape(1, BR, 128)
        he_r[...] = (he1.reshape(BR, 128, H) * e[:, :, None]).reshape(BE, H)

    full = lambda shape: pl.BlockSpec(shape, lambda i: (0, 0))
    return pl.pallas_call(
        body,
        grid=(nb,),
        in_specs=[
            pl.BlockSpec((BE, H), lambda i: (i, 0)),
            pl.BlockSpec((FE, BE), lambda i: (0, i + base)),
            pl.BlockSpec((1, BR, 128), lambda i: (i, 0, 0)),
            full((H, FE)), full((1, H)), full((H, 1)), full((1, 1)),
        ],
        out_specs=[
            pl.BlockSpec((BE, H), lambda i: (i, 0)),
            pl.BlockSpec((1, BR, 128), lambda i: (i, 0, 0)),
        ],
        out_shape=(
            jax.ShapeDtypeStruct((ne, H), jnp.float32),
            jax.ShapeDtypeStruct((nb, BR, 128), jnp.float32),
        ),
    )(xs, eat, qd128, W_pe1e, b_pe1.reshape(1, H), w2b.reshape(H, 1),
      b_pe2.reshape(1, 1))


def _scatter_l1(rows, el3d, dst3d, zrows, z1, ng):
    gpw = ng // NWORK
    @functools.partial(
        pl.kernel, mesh=_mesh(), compiler_params=_SC_PARAMS,
        out_type=(
            jax.ShapeDtypeStruct((2, NP, H), jnp.float32),
            jax.ShapeDtypeStruct((2, NP), jnp.float32),
        ),
        scratch_types=[
            pltpu.VMEM((RW, H), jnp.float32),
            pltpu.VMEM((RW, H), jnp.float32),
            pltpu.VMEM((RST, RW), jnp.int32),
            pltpu.VMEM((RST, RW), jnp.float32),
            pltpu.VMEM_SHARED((NP, H), jnp.float32),
            pltpu.VMEM_SHARED((NP,), jnp.float32),
            pltpu.SemaphoreType.DMA,
            pltpu.SemaphoreType.DMA,
            pltpu.SemaphoreType.DMA,
            pltpu.SemaphoreType.DMA,
        ])
    def k(m_h, e_h, d_h, zr_h, z1_h, cp_h, sp_h,
          mbufa, mbufb, dbuf, ebuf, csh, ssh, sema, semb, sems, semq):
        cid = lax.axis_index("c")
        sid = lax.axis_index("s")
        wid = sid * 2 + cid
        nrow = NP // 16
        pltpu.sync_copy(zr_h, csh.at[pl.ds(sid * nrow, nrow)])
        pltpu.sync_copy(z1_h, ssh.at[pl.ds(sid * nrow, nrow)])
        plsc.subcore_barrier()

        def stage(t, _):
            gidx = wid * gpw + t
            pltpu.sync_copy(d_h.at[gidx], dbuf)
            pltpu.sync_copy(e_h.at[gidx], ebuf)
            e0 = gidx * RST * RW
            pltpu.async_copy(m_h.at[pl.ds(e0, RW)], mbufa, sema)

            def scat_start(buf, j):
                pltpu.async_copy(buf, csh.at[dbuf.at[j]], sems, add=True)
                pltpu.async_copy(ebuf.at[j], ssh.at[dbuf.at[j]], semq,
                                 add=True)

            def scat_wait(buf, j):
                pltpu.make_async_copy(buf, csh.at[dbuf.at[j]], sems).wait()
                pltpu.make_async_copy(
                    ebuf.at[j], ssh.at[dbuf.at[j]], semq).wait()

            def dbl(tt, _):
                ja = 2 * tt
                jb = 2 * tt + 1
                pltpu.make_async_copy(
                    m_h.at[pl.ds(0, RW)], mbufa, sema).wait()
                pltpu.async_copy(
                    m_h.at[pl.ds(e0 + jb * RW, RW)], mbufb, semb)
                scat_start(mbufa, ja)
                pltpu.make_async_copy(
                    m_h.at[pl.ds(0, RW)], mbufb, semb).wait()
                scat_wait(mbufa, ja)
                pltpu.async_copy(
                    m_h.at[pl.ds(e0 + (jb + 1) * RW, RW)], mbufa, sema)
                scat_start(mbufb, jb)
                scat_wait(mbufb, jb)
                return 0

            lax.fori_loop(0, (RST - 1) // 2, dbl, 0)
            pltpu.make_async_copy(m_h.at[pl.ds(0, RW)], mbufa, sema).wait()
            scat_start(mbufa, RST - 1)
            scat_wait(mbufa, RST - 1)
            return 0

        lax.fori_loop(0, gpw, stage, 0)
        plsc.subcore_barrier()
        pltpu.sync_copy(csh.at[pl.ds(sid * nrow, nrow)],
                        cp_h.at[cid, pl.ds(sid * nrow, nrow)])
        pltpu.sync_copy(ssh.at[pl.ds(sid * nrow, nrow)],
                        sp_h.at[cid, pl.ds(sid * nrow, nrow)])

    return k(rows, el3d, dst3d, zrows, z1)


def _gru_block(xg, h, Wih, bih, Whh, bhh):
    gi = lax.dot_general(xg, Wih, (((1,), (1,)), ((), ())),
                         preferred_element_type=jnp.float32) + bih
    gh = lax.dot_general(h, Whh, (((1,), (1,)), ((), ())),
                         preferred_element_type=jnp.float32) + bhh
    i_r, i_z, i_n = gi[:, :H], gi[:, H:2 * H], gi[:, 2 * H:]
    h_r, h_z, h_n = gh[:, :H], gh[:, H:2 * H], gh[:, 2 * H:]
    r = jax.nn.sigmoid(i_r + h_r)
    z = jax.nn.sigmoid(i_z + h_z)
    n = jnp.tanh(i_n + r * h_n)
    return (1.0 - z) * n + z * h


def _elu(v):
    return jnp.where(v > 0, v, jnp.exp(v) - 1.0)


def _ctx(wsum, ssum, wmat_r, bias_r):
    inv = jnp.where(ssum > 0, 1.0 / jnp.maximum(ssum, 1e-30), 0.0)
    msk = (ssum > 0).astype(jnp.float32)
    w = wsum * inv
    return lax.dot_general(w, wmat_r, (((1,), (1,)), ((), ())),
                           preferred_element_type=jnp.float32) + msk * bias_r


def _gru1(wpa, wpb, spa, spb, hv, W_et, b_et, Wih1, bih1, Whh1, bhh1,
          wla, wlb):
    NBN = 5
    BN = N // NBN

    def body(wpa_r, wpb_r, spa_r, spb_r, hv_r, wet_r, bet_r,
             wih_r, bih_r, whh_r, bhh_r, wla_r, wlb_r, h1_r, qa_r, qb_r):
        wsum = wpa_r[0] + wpa_r[1] + wpb_r[0] + wpb_r[1]
        ssum = spa_r[0] + spa_r[1] + spb_r[0] + spb_r[1]
        c = _ctx(wsum, ssum, wet_r[...], bet_r[...])
        h1 = jax.nn.relu(_gru_block(_elu(c), hv_r[...], wih_r[...], bih_r[...],
                                    whh_r[...], bhh_r[...]))
        h1_r[...] = h1
        qa_r[...] = lax.dot_general(h1, wla_r[...], (((1,), (0,)), ((), ())),
                                    preferred_element_type=jnp.float32)
        qb_r[...] = lax.dot_general(h1, wlb_r[...], (((1,), (0,)), ((), ())),
                                    preferred_element_type=jnp.float32)

    full2 = lambda shape: pl.BlockSpec(shape, lambda i: (0, 0))
    return pl.pallas_call(
        body,
        grid=(NBN,),
        in_specs=[
            pl.BlockSpec((2, BN, H), lambda i: (0, i, 0)),
            pl.BlockSpec((2, BN, H), lambda i: (0, i, 0)),
            pl.BlockSpec((2, BN, 1), lambda i: (0, i, 0)),
            pl.BlockSpec((2, BN, 1), lambda i: (0, i, 0)),
            pl.BlockSpec((BN, H), lambda i: (i, 0)),
            full2((H, H)), full2((1, H)),
            full2((3 * H, H)), full2((1, 3 * H)),
            full2((3 * H, H)), full2((1, 3 * H)),
            full2((H, 1)), full2((H, 1)),
        ],
        out_specs=[
            pl.BlockSpec((BN, H), lambda i: (i, 0)),
            pl.BlockSpec((BN, 1), lambda i: (i, 0)),
            pl.BlockSpec((BN, 1), lambda i: (i, 0)),
        ],
        out_shape=(
            jax.ShapeDtypeStruct((N, H), jnp.float32),
            jax.ShapeDtypeStruct((NP, 1), jnp.float32),
            jax.ShapeDtypeStruct((NP, 1), jnp.float32),
        ),
    )(wpa, wpb, spa, spb, hv, W_et, b_et.reshape(1, H),
      Wih1, bih1.reshape(1, 3 * H), Whh1, bhh1.reshape(1, 3 * H),
      wla.reshape(H, 1), wlb.reshape(H, 1))


def _edge2_sc(qatab, qbtab, src3d, dst3d, bvec, z1):
    @functools.partial(
        pl.kernel, mesh=_mesh(), compiler_params=_SC_PARAMS,
        out_type=(
            jax.ShapeDtypeStruct((NG, RST, RW), jnp.float32),
            jax.ShapeDtypeStruct((2, NP), jnp.float32),
        ),
        scratch_types=[
            pltpu.VMEM((NP,), jnp.float32),
            pltpu.VMEM((NP,), jnp.float32),
            pltpu.VMEM((16,), jnp.float32),
            pltpu.VMEM((RST, RW), jnp.int32),
            pltpu.VMEM((RST, RW), jnp.int32),
            pltpu.VMEM((RST, RW), jnp.float32),
            pltpu.VMEM_SHARED((NP,), jnp.float32),
        ])
    def k(qa_h, qb_h, s_h, d_h, b_h, z1_h, e2_h, sp_h,
          qat, qbt, bbuf, sbuf, dbuf, obuf, ssh):
        cid = lax.axis_index("c")
        sid = lax.axis_index("s")
        wid = sid * 2 + cid
        nrow = NP // 16
        pltpu.sync_copy(qa_h, qat)
        pltpu.sync_copy(qb_h, qbt)
        pltpu.sync_copy(b_h, bbuf)
        pltpu.sync_copy(z1_h, ssh.at[pl.ds(sid * nrow, nrow)])
        plsc.subcore_barrier()
        bv = bbuf[...]

        def stage(t, _):
            gidx = wid * GPW + t
            pltpu.sync_copy(s_h.at[gidx], sbuf)
            pltpu.sync_copy(d_h.at[gidx], dbuf)

            def inner(j, _):
                for kk in range(RW // 16):
                    iv = dbuf[j, pl.ds(kk * 16, 16)]
                    iv2 = sbuf[j, pl.ds(kk * 16, 16)]
                    lv = (plsc.load_gather(qat, [iv])
                          + plsc.load_gather(qbt, [iv2]) + bv)
                    lv = jnp.where(lv >= 0, lv, 0.01 * lv)
                    obuf[j, pl.ds(kk * 16, 16)] = jnp.exp(lv)
                pltpu.sync_copy(obuf.at[j], ssh.at[dbuf.at[j]], add=True)
                return 0

            lax.fori_loop(0, RST, inner, 0)
            pltpu.sync_copy(obuf, e2_h.at[gidx])
            return 0

        lax.fori_loop(0, GPW, stage, 0)
        plsc.subcore_barrier()
        pltpu.sync_copy(ssh.at[pl.ds(sid * nrow, nrow)],
                        sp_h.at[cid, pl.ds(sid * nrow, nrow)])

    return k(qatab, qbtab, src3d, dst3d, bvec, z1)


def _gather_scale_scatter(table, src3d, e23d, dst3d, zrows):
    @functools.partial(
        pl.kernel, mesh=_mesh(), compiler_params=_SC_PARAMS,
        out_type=jax.ShapeDtypeStruct((2, NP, H), jnp.float32),
        scratch_types=[
            pltpu.VMEM((RST, RW), jnp.int32),
            pltpu.VMEM((RST, RW), jnp.int32),
            pltpu.VMEM((RST, RW), jnp.float32),
            pltpu.VMEM((RW, H), jnp.float32),
            pltpu.VMEM((RW, H), jnp.float32),
            pltpu.VMEM_SHARED((NP, H), jnp.float32),
            pltpu.SemaphoreType.DMA,
            pltpu.SemaphoreType.DMA,
            pltpu.SemaphoreType.DMA,
        ])
    def k(tab_h, s_h, e_h, d_h, zr_h, tp_h,
          sbuf, dbuf, ebuf, rbufa, rbufb, csh, sema, semb, sems):
        cid = lax.axis_index("c")
        sid = lax.axis_index("s")
        wid = sid * 2 + cid
        nrow = NP // 16
        pltpu.sync_copy(zr_h, csh.at[pl.ds(sid * nrow, nrow)])
        plsc.subcore_barrier()

        def stage(t, _):
            gidx = wid * GPW + t
            pltpu.sync_copy(s_h.at[gidx], sbuf)
            pltpu.sync_copy(d_h.at[gidx], dbuf)
            pltpu.sync_copy(e_h.at[gidx], ebuf)
            pltpu.async_copy(tab_h.at[sbuf.at[0]], rbufa, sema)

            def scale_scat(buf, j):
                for kb in range(RW // 16):
                    av16 = ebuf[j, pl.ds(kb * 16, 16)]
                    for rr in range(16):
                        av = jnp.full((16,), av16[rr], jnp.float32)
                        row = kb * 16 + rr
                        for cc in range(H // 16):
                            buf[row, pl.ds(cc * 16, 16)] = (
                                buf[row, pl.ds(cc * 16, 16)] * av)
                pltpu.async_copy(buf, csh.at[dbuf.at[j]], sems, add=True)

            def scat_wait(buf, j):
                pltpu.make_async_copy(buf, csh.at[dbuf.at[j]], sems).wait()

            def dbl(tt, _):
                ja = 2 * tt
                jb = 2 * tt + 1
                pltpu.make_async_copy(
                    tab_h.at[pl.ds(0, RW)], rbufa, sema).wait()
                pltpu.async_copy(tab_h.at[sbuf.at[jb]], rbufb, semb)
                scale_scat(rbufa, ja)
                pltpu.make_async_copy(
                    tab_h.at[pl.ds(0, RW)], rbufb, semb).wait()
                scat_wait(rbufa, ja)
                pltpu.async_copy(tab_h.at[sbuf.at[jb + 1]], rbufa, sema)
                scale_scat(rbufb, jb)
                scat_wait(rbufb, jb)
                return 0

            lax.fori_loop(0, (RST - 1) // 2, dbl, 0)
            pltpu.make_async_copy(tab_h.at[pl.ds(0, RW)], rbufa, sema).wait()
            scale_scat(rbufa, RST - 1)
            scat_wait(rbufa, RST - 1)
            return 0

        lax.fori_loop(0, GPW, stage, 0)
        plsc.subcore_barrier()
        pltpu.sync_copy(csh.at[pl.ds(sid * nrow, nrow)],
                        tp_h.at[cid, pl.ds(sid * nrow, nrow)])

    return k(table, src3d, e23d, dst3d, zrows)


def _gru2_pool(tp, s2p, h1, W_pn2, b_pn2, Wih2, bih2, Whh2, bhh2, gids):
    NBN = 10
    BN = N // NBN

    def body(tp_r, sp_r, h1_r, wpn_r, bpn_r, wih_r, bih_r, whh_r, bhh_r,
             gid_r, out_r, cnt_r):
        i = pl.program_id(0)

        @pl.when(i == 0)
        def _():
            out_r[...] = jnp.zeros_like(out_r)
            cnt_r[...] = jnp.zeros_like(cnt_r)

        c = _ctx(tp_r[0] + tp_r[1], sp_r[0] + sp_r[1], wpn_r[...], bpn_r[...])
        h2 = jax.nn.relu(_gru_block(_elu(c), h1_r[...], wih_r[...], bih_r[...],
                                    whh_r[...], bhh_r[...]))
        onehot = (gid_r[...] == lax.broadcasted_iota(jnp.int32, (BN, G), 1)
                  ).astype(jnp.float32)
        out_r[...] += lax.dot_general(onehot, h2, (((0,), (0,)), ((), ())),
                                      preferred_element_type=jnp.float32)
        cnt_r[...] += lax.dot_general(
            onehot, jnp.ones((BN, 1), jnp.float32),
            (((0,), (0,)), ((), ())), preferred_element_type=jnp.float32)

        @pl.when(i == NBN - 1)
        def _():
            out_r[...] = out_r[...] / jnp.maximum(cnt_r[...], 1.0)

    full2 = lambda shape: pl.BlockSpec(shape, lambda i: (0, 0))
    return pl.pallas_call(
        body,
        grid=(NBN,),
        in_specs=[
            pl.BlockSpec((2, BN, H), lambda i: (0, i, 0)),
            pl.BlockSpec((2, BN, 1), lambda i: (0, i, 0)),
            pl.BlockSpec((BN, H), lambda i: (i, 0)),
            full2((H, H)), full2((1, H)),
            full2((3 * H, H)), full2((1, 3 * H)),
            full2((3 * H, H)), full2((1, 3 * H)),
            pl.BlockSpec((BN, 1), lambda i: (i, 0)),
        ],
        out_specs=pl.BlockSpec((G, H), lambda i: (0, 0)),
        out_shape=jax.ShapeDtypeStruct((G, H), jnp.float32),
        scratch_shapes=[pltpu.VMEM((G, 1), jnp.float32)],
    )(tp, s2p, h1, W_pn2, b_pn2.reshape(1, H),
      Wih2, bih2.reshape(1, 3 * H), Whh2, bhh2.reshape(1, 3 * H),
      gids.reshape(N, 1))


def kernel(x, edge_attr, edge_index, graph_ids,
           W_pn, b_pn, W_pe1, b_pe1, W_pe2, b_pe2, W_et, b_et,
           Wih1, bih1, Whh1, bhh1,
           W_pe_l, b_pe_l, W_pn2, b_pn2, Wih2, bih2, Whh2, bhh2):
    src3d = edge_index[0].reshape(NG, RST, RW)
    dst3d = edge_index[1].reshape(NG, RST, RW)

    W_pe1x = W_pe1[:, :FN]
    W_pe1e = W_pe1[:, FN:]
    w2a = W_pe2[0, :H]
    w2b = W_pe2[0, H:]
    wla = W_pe_l[0, :H]
    wlb = W_pe_l[0, H:]

    z1 = jnp.zeros((NP // 16,), jnp.float32)
    zrows = jnp.zeros((NP // 16, H), jnp.float32)

    NGA = 96
    NBA = NGA * RST * RW // BE
    NBB = (NG - NGA) * RST * RW // BE
    hv, xp, q = _node_prep(x, W_pn, b_pn, W_pe1x, w2a)
    qp = q.reshape(NP)
    eat = edge_attr.T
    srca, dsta = src3d[:NGA], dst3d[:NGA]
    srcb, dstb = src3d[NGA:], dst3d[NGA:]
    xsa, qda = _gather1(xp, srca, dsta, qp, NGA)
    xsb, qdb = _gather1(xp, srcb, dstb, qp, NG - NGA)
    hea, ela = _edge1(xsa, eat, qda.reshape(NBA, BE // 128, 128),
                      W_pe1e, b_pe1, w2b, b_pe2, NBA, 0)
    heb, elb = _edge1(xsb, eat, qdb.reshape(NBB, BE // 128, 128),
                      W_pe1e, b_pe1, w2b, b_pe2, NBB, NBA)
    wpa, spa = _scatter_l1(hea, ela.reshape(NGA, RST, RW), dsta,
                           zrows, z1, NGA)
    wpb, spb = _scatter_l1(heb, elb.reshape(NG - NGA, RST, RW), dstb,
                           zrows, z1, NG - NGA)
    h1, qa, qb = _gru1(wpa, wpb, spa.reshape(2, NP, 1), spb.reshape(2, NP, 1),
                       hv, W_et, b_et, Wih1, bih1, Whh1, bhh1, wla, wlb)

    bl = jnp.full((16,), b_pe_l[0], jnp.float32)
    e23d, s2p = _edge2_sc(qa.reshape(NP), qb.reshape(NP), src3d, dst3d, bl, z1)
    tp = _gather_scale_scatter(h1, src3d, e23d, dst3d, zrows)
    return _gru2_pool(tp, s2p.reshape(2, NP, 1), h1, W_pn2, b_pn2,
                      Wih2, bih2, Whh2, bhh2, graph_ids)
